# baseline (speedup 1.0000x reference)
import functools

import jax
import jax.numpy as jnp
from jax import lax
from jax.experimental import pallas as pl
from jax.experimental.pallas import tpu as pltpu
from jax.experimental.pallas import tpu_sc as plsc

NC, NS, L = 2, 16, 16
NW = NC * NS

CHUNK = 128


def _sc_body(H, rows_per_w, f_hbm, g_hbm, out_hbm, gbuf, fbuf):
    wid = lax.axis_index("s") * NC + lax.axis_index("c")
    base = wid * rows_per_w
    pltpu.sync_copy(g_hbm.at[pl.ds(base * 1, rows_per_w)], gbuf)

    nchunks = rows_per_w // CHUNK
    nvec = H // L

    def chunk_body(c, _):
        row0 = base + c * CHUNK
        pltpu.sync_copy(f_hbm.at[pl.ds(row0, CHUNK)], fbuf)

        def q_body(q, _):
            g16 = gbuf[pl.ds(c * CHUNK + q * L, L)]
            s16 = 1.0 / jnp.maximum(g16, 1).astype(jnp.float32)
            for r in range(L):
                s = s16[r]
                row = q * L + r
                for v in range(nvec):
                    sl = pl.ds(v * L, L)
                    fbuf[row, sl] = fbuf[row, sl] * s
            return 0

        lax.fori_loop(0, CHUNK // L, q_body, 0)
        pltpu.sync_copy(fbuf, out_hbm.at[pl.ds(row0, CHUNK)])
        return 0

    lax.fori_loop(0, nchunks, chunk_body, 0)


def kernel(feats, groups):
    B, S, H = feats.shape
    G = groups.shape[1]
    rows = B * S
    rows_per_w = rows // NW

    f2 = feats.reshape(rows, H)
    g1 = groups.reshape(rows)

    mesh = plsc.VectorSubcoreMesh(core_axis_name="c", subcore_axis_name="s")
    sc_call = pl.kernel(
        functools.partial(_sc_body, H, rows_per_w),
        out_type=jax.ShapeDtypeStruct((rows, H), feats.dtype),
        mesh=mesh,
        scratch_types=[
            pltpu.VMEM((rows_per_w,), jnp.int32),
            pltpu.VMEM((CHUNK, H), jnp.float32),
        ],
    )
    out = sc_call(f2, g1)

    agg_feats = out.reshape(B, G, H)
    group_lengths = jnp.full((B,), G, dtype=jnp.int32)
    return agg_feats, group_lengths


# speedup vs baseline: 1.4132x; 1.4132x over previous
import functools

import jax
import jax.numpy as jnp
from jax import lax
from jax.experimental import pallas as pl
from jax.experimental.pallas import tpu as pltpu
from jax.experimental.pallas import tpu_sc as plsc

NC, NS, L = 2, 16, 16
NW = NC * NS

CHUNK = 64


def _sc_body(H, rows_per_w, f_hbm, g_hbm, out_hbm, gbuf,
             fin0, fin1, fout0, fout1, si0, si1, so0, so1):
    wid = lax.axis_index("s") * NC + lax.axis_index("c")
    base = wid * rows_per_w
    pltpu.sync_copy(g_hbm.at[pl.ds(base, rows_per_w)], gbuf)

    fins, fouts = (fin0, fin1), (fout0, fout1)
    sins, souts = (si0, si1), (so0, so1)
    nchunks = rows_per_w // CHUNK
    npairs = nchunks // 2
    nvec = H // L

    def in_copy(b, c):
        return pltpu.make_async_copy(
            f_hbm.at[pl.ds(base + c * CHUNK, CHUNK)], fins[b], sins[b])

    def out_copy(b, c):
        return pltpu.make_async_copy(
            fouts[b], out_hbm.at[pl.ds(base + c * CHUNK, CHUNK)], souts[b])

    in_copy(0, 0).start()
    in_copy(1, 1).start()

    def pair_body(t, _):
        for b in range(2):
            c = 2 * t + b
            in_copy(b, c).wait()

            @pl.when(t > 0)
            def _():
                out_copy(b, c - 2).wait()

            def q_body(q, _):
                g16 = gbuf[pl.ds(c * CHUNK + q * L, L)]
                s16 = 1.0 / jnp.maximum(g16, 1).astype(jnp.float32)
                for r in range(L):
                    s = s16[r]
                    row = q * L + r
                    for v in range(nvec):
                        sl = pl.ds(v * L, L)
                        fouts[b][row, sl] = fins[b][row, sl] * s
                return 0

            lax.fori_loop(0, CHUNK // L, q_body, 0)
            start_out = out_copy(b, c)
            start_out.start()

            @pl.when(t < npairs - 1)
            def _():
                in_copy(b, c + 2).start()
        return 0

    lax.fori_loop(0, npairs, pair_body, 0)
    out_copy(0, nchunks - 2).wait()
    out_copy(1, nchunks - 1).wait()


def kernel(feats, groups):
    B, S, H = feats.shape
    G = groups.shape[1]
    rows = B * S
    rows_per_w = rows // NW

    f2 = feats.reshape(rows, H)
    g1 = groups.reshape(rows)

    mesh = plsc.VectorSubcoreMesh(core_axis_name="c", subcore_axis_name="s")
    sc_call = pl.kernel(
        functools.partial(_sc_body, H, rows_per_w),
        out_type=jax.ShapeDtypeStruct((rows, H), feats.dtype),
        mesh=mesh,
        scratch_types=[
            pltpu.VMEM((rows_per_w,), jnp.int32),
            pltpu.VMEM((CHUNK, H), jnp.float32),
            pltpu.VMEM((CHUNK, H), jnp.float32),
            pltpu.VMEM((CHUNK, H), jnp.float32),
            pltpu.VMEM((CHUNK, H), jnp.float32),
            pltpu.SemaphoreType.DMA,
            pltpu.SemaphoreType.DMA,
            pltpu.SemaphoreType.DMA,
            pltpu.SemaphoreType.DMA,
        ],
    )
    out = sc_call(f2, g1)

    agg_feats = out.reshape(B, G, H)
    group_lengths = jnp.full((B,), G, dtype=jnp.int32)
    return agg_feats, group_lengths
